# Initial kernel scaffold; baseline (speedup 1.0000x reference)
import jax
import jax.numpy as jnp
from jax.experimental import pallas as pl


def kernel(x, edge_index, edge_weight, pos, subgraph_assignment, emb_table, emb_gn_w, emb_gn_b, W0, b0, W1, b1, gn0_w, gn0_b, gnL_w, gnL_b, mlp_w, mlp_b):
    raise NotImplementedError("write your pallas kernel here")


# speedup vs baseline: 5.8064x; 5.8064x over previous
import functools

import jax
import jax.numpy as jnp
from jax import lax
from jax.experimental import pallas as pl
from jax.experimental.pallas import tpu as pltpu
from jax.experimental.pallas import tpu_sc as plsc

N = 10000
E = 320000
D = 128
HID = 128
NCLUS = 16
NSUB = 100

NC = 2
NS = 16
LANES = 16
EDGES_PER_WORKER = E // (NC * NS)
CHUNK = 80
NCHUNK = EDGES_PER_WORKER // CHUNK
ROWS_LO = 624
ROWS_HI = N - 15 * ROWS_LO


def _barrier():
  plsc.subcore_barrier()


def _row_chunks(total, step):
  out = []
  off = 0
  while off < total:
    sz = min(step, total - off)
    out.append((off, sz))
    off += sz
  return out


def _gather_rows(hp_hbm, col_v, rows_v, sem):
  pltpu.async_copy(hp_hbm.at[col_v], rows_v, sem).wait()


def _scatter_add_rows(rows_v, acc_sh, row_v):
  pltpu.sync_copy(rows_v, acc_sh.at[row_v], add=True)


def _scatter_add_deg(ew_v, deg_sh, row_v):
  pltpu.sync_copy(ew_v.at[pl.ds(0, CHUNK)], deg_sh.at[row_v], add=True)


def _sc_agg_body(with_deg, hp_hbm, row_hbm, col_hbm, ew_hbm,
                 part_out, degp_out, col_v, row_v, ew_v, rows_v, zbuf,
                 zdeg, acc_sh, deg_sh, sem):
  c = lax.axis_index("c")
  s = lax.axis_index("s")

  def zrow(i, _):
    for f in range(D // LANES):
      zbuf[i, pl.ds(f * LANES, LANES)] = jnp.zeros((LANES,), jnp.float32)
    return 0
  lax.fori_loop(0, CHUNK, zrow, 0)
  for f in range(ROWS_HI // LANES):
    zdeg[pl.ds(f * LANES, LANES)] = jnp.zeros((LANES,), jnp.float32)

  @pl.when(s < NS - 1)
  def _():
    for off, sz in _row_chunks(ROWS_LO, CHUNK):
      pltpu.sync_copy(zbuf.at[pl.ds(0, sz)],
                      acc_sh.at[pl.ds(s * ROWS_LO + off, sz)])
    if with_deg:
      pltpu.sync_copy(zdeg.at[pl.ds(0, ROWS_LO)],
                      deg_sh.at[pl.ds(s * ROWS_LO, ROWS_LO)])

  @pl.when(s == NS - 1)
  def _():
    for off, sz in _row_chunks(ROWS_HI, CHUNK):
      pltpu.sync_copy(zbuf.at[pl.ds(0, sz)],
                      acc_sh.at[pl.ds((NS - 1) * ROWS_LO + off, sz)])
    if with_deg:
      pltpu.sync_copy(zdeg, deg_sh.at[pl.ds((NS - 1) * ROWS_LO, ROWS_HI)])

  _barrier()

  wbase = (c * NS + s) * EDGES_PER_WORKER

  def step(i, _):
    base = wbase + i * CHUNK
    pltpu.sync_copy(col_hbm.at[pl.ds(base, CHUNK)], col_v)
    pltpu.sync_copy(row_hbm.at[pl.ds(base, CHUNK)], row_v)
    pltpu.sync_copy(ew_hbm.at[pl.ds(base, CHUNK)], ew_v.at[pl.ds(0, CHUNK)])
    _gather_rows(hp_hbm, col_v, rows_v, sem)
    if with_deg:
      _scatter_add_deg(ew_v, deg_sh, row_v)

    def scale(j, _):
      wj = ew_v[pl.ds(j, LANES)][0]
      for f in range(D // LANES):
        sl = pl.ds(f * LANES, LANES)
        rows_v[j, sl] = rows_v[j, sl] * wj
      return 0
    lax.fori_loop(0, CHUNK, scale, 0)
    _scatter_add_rows(rows_v, acc_sh, row_v)
    return 0

  lax.fori_loop(0, NCHUNK, step, 0)
  _barrier()

  @pl.when(s < NS - 1)
  def _():
    for off, sz in _row_chunks(ROWS_LO, CHUNK):
      o = s * ROWS_LO + off
      pltpu.sync_copy(acc_sh.at[pl.ds(o, sz)], zbuf.at[pl.ds(0, sz)])
      pltpu.sync_copy(zbuf.at[pl.ds(0, sz)], part_out.at[c, pl.ds(o, sz)])
    if with_deg:
      pltpu.sync_copy(deg_sh.at[pl.ds(s * ROWS_LO, ROWS_LO)],
                      zdeg.at[pl.ds(0, ROWS_LO)])
      pltpu.sync_copy(zdeg.at[pl.ds(0, ROWS_LO)],
                      degp_out.at[pl.ds(c * N + s * ROWS_LO, ROWS_LO)])

  @pl.when(s == NS - 1)
  def _():
    for off, sz in _row_chunks(ROWS_HI, CHUNK):
      o = (NS - 1) * ROWS_LO + off
      pltpu.sync_copy(acc_sh.at[pl.ds(o, sz)], zbuf.at[pl.ds(0, sz)])
      pltpu.sync_copy(zbuf.at[pl.ds(0, sz)], part_out.at[c, pl.ds(o, sz)])
    if with_deg:
      pltpu.sync_copy(deg_sh.at[pl.ds((NS - 1) * ROWS_LO, ROWS_HI)], zdeg)
      pltpu.sync_copy(zdeg,
                      degp_out.at[pl.ds(c * N + (NS - 1) * ROWS_LO, ROWS_HI)])


def _make_sc_agg(with_deg, interpret=False):
  mesh = plsc.VectorSubcoreMesh(core_axis_name="c", subcore_axis_name="s",
                                num_cores=NC, num_subcores=NS)
  out_type = [jax.ShapeDtypeStruct((NC, N, D), jnp.float32)]
  if with_deg:
    out_type.append(jax.ShapeDtypeStruct((NC * N,), jnp.float32))
  scratch = [
      pltpu.VMEM((CHUNK,), jnp.int32),
      pltpu.VMEM((CHUNK,), jnp.int32),
      pltpu.VMEM((CHUNK + LANES,), jnp.float32),
      pltpu.VMEM((CHUNK, D), jnp.float32),
      pltpu.VMEM((CHUNK, D), jnp.float32),
      pltpu.VMEM((ROWS_HI,), jnp.float32),
      pltpu.VMEM_SHARED((N, D), jnp.float32),
      pltpu.VMEM_SHARED((N,), jnp.float32),
      pltpu.SemaphoreType.DMA,
  ]

  if with_deg:
    def body(hp, row, col, ew, part, degp, *scr):
      _sc_agg_body(True, hp, row, col, ew, part, degp, *scr)
  else:
    def body(hp, row, col, ew, part, *scr):
      _sc_agg_body(False, hp, row, col, ew, part, None, *scr)

  return pl.kernel(body, out_type=out_type, mesh=mesh,
                   scratch_types=scratch, interpret=interpret)


def _tc1_body(x_ref, emb_ref, gw_ref, gb_ref, w0_ref, b0_ref, hp_ref):
  iota = lax.broadcasted_iota(jnp.int32, (N, D), 1)
  onehot = (iota == x_ref[...]).astype(jnp.float32)
  h = jnp.dot(onehot, emb_ref[...], preferred_element_type=jnp.float32,
              precision=lax.Precision.HIGHEST)
  mean = jnp.mean(h, axis=0, keepdims=True)
  var = jnp.mean((h - mean) ** 2, axis=0, keepdims=True)
  hn = gw_ref[...] * (h - mean) / jnp.sqrt(var + 1e-5) + gb_ref[...]
  z = lax.dot_general(hn.astype(jnp.bfloat16), w0_ref[...].astype(jnp.bfloat16),
                      (((1,), (1,)), ((), ())),
                      preferred_element_type=jnp.float32)
  hp_ref[...] = jnp.maximum(z + b0_ref[...], 0.0)


def _tc2_body(part_ref, degp_ref, gw_ref, gb_ref, w1_ref, b1_ref,
              h1_ref, rdeg_ref, hp_ref):
  deg = degp_ref[pl.ds(0, N)]
  for cc in range(1, NC):
    deg = deg + degp_ref[pl.ds(cc * N, N)]
  deg = jnp.where(deg < 0.5, deg + 1.0, deg)
  rdeg = 1.0 / deg
  rdeg_ref[...] = rdeg[:, None]
  acc = jnp.sum(part_ref[...], axis=0)
  h1 = acc * rdeg[:, None]
  h1_ref[...] = h1
  mean = jnp.mean(h1, axis=0, keepdims=True)
  var = jnp.mean((h1 - mean) ** 2, axis=0, keepdims=True)
  hn = gw_ref[...] * (h1 - mean) / jnp.sqrt(var + 1e-5) + gb_ref[...]
  hn = jnp.maximum(hn, 0.0)
  z = lax.dot_general(hn.astype(jnp.bfloat16), w1_ref[...].astype(jnp.bfloat16),
                      (((1,), (1,)), ((), ())),
                      preferred_element_type=jnp.float32)
  hp_ref[...] = jnp.maximum(z + b1_ref[...], 0.0)


def _tc3a_body(part_ref, rdeg_ref, h2_ref):
  h2_ref[...] = jnp.sum(part_ref[...], axis=0) * rdeg_ref[...]


def _tc3_body(h1_ref, h2_ref, gw_ref, gb_ref, mw_ref, mb_ref,
              sga_ref, s_ref, oh_ref, stc_ref):
  def gn_half(h, off):
    mean = jnp.mean(h, axis=0, keepdims=True)
    var = jnp.mean((h - mean) ** 2, axis=0, keepdims=True)
    gw = gw_ref[pl.ds(off, HID)]
    gb = gb_ref[pl.ds(off, HID)]
    return gw * (h - mean) / jnp.sqrt(var + 1e-5) + gb

  xn1 = gn_half(h1_ref[...], 0)
  xn2 = gn_half(h2_ref[...], HID)
  s = (lax.dot_general(xn1.astype(jnp.bfloat16),
                       mw_ref[:, pl.ds(0, HID)].astype(jnp.bfloat16),
                       (((1,), (1,)), ((), ())),
                       preferred_element_type=jnp.float32)
       + lax.dot_general(xn2.astype(jnp.bfloat16),
                         mw_ref[:, pl.ds(HID, HID)].astype(jnp.bfloat16),
                         (((1,), (1,)), ((), ())),
                         preferred_element_type=jnp.float32)
       + mb_ref[...])
  s_ref[...] = s
  mx = jnp.max(s, axis=1, keepdims=True)
  iota = lax.broadcasted_iota(jnp.int32, (N, NCLUS), 1)
  cand = jnp.where(s == mx, iota, NCLUS)
  am = jnp.min(cand, axis=1, keepdims=True)
  oh_ref[...] = (iota == am).astype(jnp.float32)
  p = jnp.exp(s - mx)
  p = p / jnp.sum(p, axis=1, keepdims=True)
  denom = jnp.maximum(jnp.sum(p, axis=0, keepdims=True), 1e-12)
  p_norm = p / denom
  stc_ref[...] = lax.dot_general(
      p_norm.astype(jnp.bfloat16), sga_ref[...].astype(jnp.bfloat16),
      (((0,), (1,)), ((), ())), preferred_element_type=jnp.float32)


def kernel(x, edge_index, edge_weight, pos, subgraph_assignment, emb_table,
           emb_gn_w, emb_gn_b, W0, b0, W1, b1, gn0_w, gn0_b, gnL_w, gnL_b,
           mlp_w, mlp_b):
  row = edge_index[0].astype(jnp.int32)
  col = edge_index[1].astype(jnp.int32)
  emb_pad = jnp.zeros((D, D), jnp.float32).at[: emb_table.shape[0]].set(emb_table)
  x2 = x.astype(jnp.int32).reshape(N, 1)

  hp0 = pl.pallas_call(
      _tc1_body,
      out_shape=jax.ShapeDtypeStruct((N, HID), jnp.float32),
  )(x2, emb_pad, emb_gn_w, emb_gn_b, W0, b0)

  part0, degp = _make_sc_agg(True)(hp0, row, col, edge_weight)

  h1, rdeg, hp1 = pl.pallas_call(
      _tc2_body,
      out_shape=[
          jax.ShapeDtypeStruct((N, HID), jnp.float32),
          jax.ShapeDtypeStruct((N, 1), jnp.float32),
          jax.ShapeDtypeStruct((N, HID), jnp.float32),
      ],
  )(part0, degp, gn0_w, gn0_b, W1, b1)

  (part1,) = _make_sc_agg(False)(hp1, row, col, edge_weight)

  h2 = pl.pallas_call(
      _tc3a_body,
      out_shape=jax.ShapeDtypeStruct((N, HID), jnp.float32),
  )(part1, rdeg)

  s, updated_s, stc = pl.pallas_call(
      _tc3_body,
      out_shape=[
          jax.ShapeDtypeStruct((N, NCLUS), jnp.float32),
          jax.ShapeDtypeStruct((N, NCLUS), jnp.float32),
          jax.ShapeDtypeStruct((NCLUS, NSUB), jnp.float32),
      ],
  )(h1, h2, gnL_w, gnL_b, mlp_w, mlp_b, subgraph_assignment)

  return (s, updated_s, stc)
